# baseline (speedup 1.0000x reference)
import jax
import jax.numpy as jnp
from jax import lax
from jax.experimental import pallas as pl
from jax.experimental.pallas import tpu as pltpu
from jax.experimental.pallas import tpu_sc as plsc

N_NODES = 10000
N_EDGES = 160000
D = 256
DH = 128
NC, NS = 2, 16
EDGES_PER_TILE = N_EDGES // NS
CHUNK = 40
NCHUNK = EDGES_PER_TILE // CHUNK
N_PAD = 10112
ROWS_PER_TILE = N_PAD // NS
NBUF = 4


def _agg_body(xs_hbm, srco_hbm, dst_hbm, ea_hbm, zero_hbm, agg_hbm,
              accum, idx_v, dst_v, xr_v, ea_v, sem_idx, sem_data, sem_sc):
    c = lax.axis_index("c")
    s = lax.axis_index("s")

    r0 = s * ROWS_PER_TILE
    pltpu.sync_copy(zero_hbm.at[pl.ds(r0, ROWS_PER_TILE)],
                    accum.at[pl.ds(r0, ROWS_PER_TILE)])
    plsc.subcore_barrier()

    ebase = s * EDGES_PER_TILE

    def issue_loads(j, b):
        e0 = ebase + j * CHUNK
        pltpu.async_copy(srco_hbm.at[pl.ds(c * N_EDGES + e0, CHUNK)],
                         idx_v[b], sem_idx[b])
        pltpu.async_copy(dst_hbm.at[pl.ds(e0, CHUNK)], dst_v[b], sem_idx[b])
        pltpu.async_copy(ea_hbm.at[pl.ds(e0, CHUNK), pl.ds(c * DH, DH)],
                         ea_v[b], sem_data[b])

    def wait_idx(b):
        pltpu.make_async_copy(srco_hbm.at[pl.ds(0, CHUNK)], idx_v[b],
                              sem_idx[b]).wait()
        pltpu.make_async_copy(dst_hbm.at[pl.ds(0, CHUNK)], dst_v[b],
                              sem_idx[b]).wait()

    def issue_gather(b):
        pltpu.async_copy(xs_hbm.at[idx_v[b]], xr_v[b], sem_data[b])

    def wait_data(b):
        pltpu.make_async_copy(ea_hbm.at[pl.ds(0, CHUNK), pl.ds(0, DH)],
                              ea_v[b], sem_data[b]).wait()
        pltpu.make_async_copy(xs_hbm.at[idx_v[b]], xr_v[b],
                              sem_data[b]).wait()

    def compute(b):
        xr, ea = xr_v[b], ea_v[b]

        def row_body(r, rcarry):
            for cc in range(DH // 16):
                v = xr[r, pl.ds(cc * 16, 16)] + ea[r, pl.ds(cc * 16, 16)]
                xr[r, pl.ds(cc * 16, 16)] = jnp.maximum(v, 0.0)
            return rcarry

        lax.fori_loop(0, CHUNK, row_body, 0)

    def issue_scatter(b):
        pltpu.async_copy(xr_v[b], accum.at[dst_v[b]], sem_sc[b], add=True)

    def wait_scatter(b):
        pltpu.make_async_copy(xr_v[b], accum.at[dst_v[b]], sem_sc[b]).wait()

    for b in range(3):
        issue_loads(b, b)
    wait_idx(0)
    issue_gather(0)

    for j in range(NBUF):
        b = j % NBUF
        bg = (b + 1) % NBUF
        wait_idx(bg)
        issue_gather(bg)
        wait_data(b)
        compute(b)
        issue_scatter(b)
        if j >= 1:
            wait_scatter((b + NBUF - 1) % NBUF)
        issue_loads(j + 3, (b + 3) % NBUF)

    NBODY = (NCHUNK - NBUF - 1) // NBUF

    def body(i, carry):
        j0 = NBUF + i * NBUF
        for u in range(NBUF):
            j = j0 + u
            b = u
            bg = (b + 1) % NBUF
            wait_idx(bg)
            issue_gather(bg)
            wait_data(b)
            compute(b)
            issue_scatter(b)
            wait_scatter((b + NBUF - 1) % NBUF)
            pl.when(j <= NCHUNK - NBUF)(
                lambda jn=j + 3, bn=(b + 3) % NBUF: issue_loads(jn, bn))
        return carry

    lax.fori_loop(0, NBODY, body, 0)

    for j in range(NBUF + NBODY * NBUF, NCHUNK):
        b = j % NBUF
        if j < NCHUNK - 1:
            bg = (b + 1) % NBUF
            wait_idx(bg)
            issue_gather(bg)
        wait_data(b)
        compute(b)
        issue_scatter(b)
        wait_scatter((b + NBUF - 1) % NBUF)
        if j + 3 <= NCHUNK - 1:
            issue_loads(j + 3, (b + 3) % NBUF)
    wait_scatter((NCHUNK - 1) % NBUF)

    plsc.subcore_barrier()

    pltpu.sync_copy(accum.at[pl.ds(r0, ROWS_PER_TILE)],
                    agg_hbm.at[c, pl.ds(r0, ROWS_PER_TILE)])


def _sc_agg(xs, srco, dst, ea, zeros):
    mesh = plsc.VectorSubcoreMesh(core_axis_name="c", subcore_axis_name="s")
    return pl.kernel(
        _agg_body,
        out_type=jax.ShapeDtypeStruct((NC, N_PAD, DH), jnp.float32),
        mesh=mesh,
        scratch_types=[
            pltpu.VMEM_SHARED((N_PAD, DH), jnp.float32),
            [pltpu.VMEM((CHUNK,), jnp.int32) for _ in range(NBUF)],
            [pltpu.VMEM((CHUNK,), jnp.int32) for _ in range(NBUF)],
            [pltpu.VMEM((CHUNK, DH), jnp.float32) for _ in range(NBUF)],
            [pltpu.VMEM((CHUNK, DH), jnp.float32) for _ in range(NBUF)],
            [pltpu.SemaphoreType.DMA for _ in range(NBUF)],
            [pltpu.SemaphoreType.DMA for _ in range(NBUF)],
            [pltpu.SemaphoreType.DMA for _ in range(NBUF)],
        ],
    )(xs, srco, dst, ea, zeros)


ROW_BLK = 2000


def _mlp_body(eps_ref, x_ref, a0_ref, a1_ref, w1_ref, b1_ref, w2_ref, b2_ref,
              out_ref):
    h = x_ref[...] * (1.0 + eps_ref[0]) + jnp.concatenate(
        [a0_ref[...], a1_ref[...]], axis=1)
    h1 = lax.dot_general(h.astype(jnp.bfloat16),
                         w1_ref[...].astype(jnp.bfloat16),
                         (((1,), (1,)), ((), ())),
                         preferred_element_type=jnp.float32)
    h1 = jnp.maximum(h1 + b1_ref[...], 0.0)
    out_ref[...] = lax.dot_general(h1.astype(jnp.bfloat16),
                                   w2_ref[...].astype(jnp.bfloat16),
                                   (((1,), (1,)), ((), ())),
                                   preferred_element_type=jnp.float32) + b2_ref[...]


def _mlp(x, a0, a1, W1, b1, W2, b2, eps):
    grid = (N_NODES // ROW_BLK,)
    return pl.pallas_call(
        _mlp_body,
        grid=grid,
        in_specs=[
            pl.BlockSpec(memory_space=pltpu.SMEM),
            pl.BlockSpec((ROW_BLK, D), lambda i: (i, 0)),
            pl.BlockSpec((ROW_BLK, DH), lambda i: (i, 0)),
            pl.BlockSpec((ROW_BLK, DH), lambda i: (i, 0)),
            pl.BlockSpec((D, D), lambda i: (0, 0)),
            pl.BlockSpec((1, D), lambda i: (0, 0)),
            pl.BlockSpec((D, D), lambda i: (0, 0)),
            pl.BlockSpec((1, D), lambda i: (0, 0)),
        ],
        out_specs=pl.BlockSpec((ROW_BLK, D), lambda i: (i, 0)),
        out_shape=jax.ShapeDtypeStruct((N_NODES, D), jnp.float32),
    )(eps, x, a0, a1, W1, b1, W2, b2)


def kernel(x, edge_index, edge_attr, W1, b1, W2, b2, eps):
    src = edge_index[0].astype(jnp.int32)
    dst = edge_index[1].astype(jnp.int32)
    xs = jnp.concatenate([x[:, :DH], x[:, DH:]], axis=0)
    srco = jnp.concatenate([src, src + N_NODES])
    zeros = jnp.zeros((N_PAD, DH), jnp.float32)
    agg = _sc_agg(xs, srco, dst, edge_attr, zeros)
    eps_s = jnp.reshape(eps, (1,)).astype(jnp.float32)
    return _mlp(x, agg[0, :N_NODES], agg[1, :N_NODES], W1,
                jnp.reshape(b1, (1, D)), W2, jnp.reshape(b2, (1, D)), eps_s)


# speedup vs baseline: 1.0419x; 1.0419x over previous
import jax
import jax.numpy as jnp
from jax import lax
from jax.experimental import pallas as pl
from jax.experimental.pallas import tpu as pltpu
from jax.experimental.pallas import tpu_sc as plsc

N_NODES = 10000
N_EDGES = 160000
D = 256
DH = 128
NC, NS = 2, 16
EDGES_PER_TILE = N_EDGES // NS
CHUNK = 40
NCHUNK = EDGES_PER_TILE // CHUNK
N_PAD = 10112
ROWS_PER_TILE = N_PAD // NS
NBUF = 4


def _agg_body(xs_hbm, srco_hbm, dst_hbm, ea_hbm, zero_hbm, agg_hbm,
              accum, idx_v, dst_v, xr_v, ea_v, sem_idx, sem_data, sem_sc):
    c = lax.axis_index("c")
    s = lax.axis_index("s")

    r0 = s * ROWS_PER_TILE
    pltpu.sync_copy(zero_hbm, accum.at[pl.ds(r0, ROWS_PER_TILE)])
    plsc.subcore_barrier()

    ebase = s * EDGES_PER_TILE

    def issue_loads(j, b):
        e0 = ebase + j * CHUNK
        pltpu.async_copy(srco_hbm.at[pl.ds(c * N_EDGES + e0, CHUNK)],
                         idx_v[b], sem_idx[b])
        pltpu.async_copy(dst_hbm.at[pl.ds(e0, CHUNK)], dst_v[b], sem_idx[b])
        pltpu.async_copy(ea_hbm.at[pl.ds(e0, CHUNK), pl.ds(c * DH, DH)],
                         ea_v[b], sem_data[b])

    def wait_idx(b):
        pltpu.make_async_copy(srco_hbm.at[pl.ds(0, CHUNK)], idx_v[b],
                              sem_idx[b]).wait()
        pltpu.make_async_copy(dst_hbm.at[pl.ds(0, CHUNK)], dst_v[b],
                              sem_idx[b]).wait()

    def issue_gather(b):
        pltpu.async_copy(xs_hbm.at[idx_v[b]], xr_v[b], sem_data[b])

    def wait_data(b):
        pltpu.make_async_copy(ea_hbm.at[pl.ds(0, CHUNK), pl.ds(0, DH)],
                              ea_v[b], sem_data[b]).wait()
        pltpu.make_async_copy(xs_hbm.at[idx_v[b]], xr_v[b],
                              sem_data[b]).wait()

    def compute(b):
        xr, ea = xr_v[b], ea_v[b]

        def row_body(r, rcarry):
            for cc in range(DH // 16):
                v = xr[r, pl.ds(cc * 16, 16)] + ea[r, pl.ds(cc * 16, 16)]
                xr[r, pl.ds(cc * 16, 16)] = jnp.maximum(v, 0.0)
            return rcarry

        lax.fori_loop(0, CHUNK, row_body, 0)

    def issue_scatter(b):
        pltpu.async_copy(xr_v[b], accum.at[dst_v[b]], sem_sc[b], add=True)

    def wait_scatter(b):
        pltpu.make_async_copy(xr_v[b], accum.at[dst_v[b]], sem_sc[b]).wait()

    for b in range(3):
        issue_loads(b, b)
    wait_idx(0)
    issue_gather(0)

    for j in range(NBUF):
        b = j % NBUF
        bg = (b + 1) % NBUF
        wait_idx(bg)
        issue_gather(bg)
        wait_data(b)
        compute(b)
        issue_scatter(b)
        if j >= 1:
            wait_scatter((b + NBUF - 1) % NBUF)
        issue_loads(j + 3, (b + 3) % NBUF)

    NBODY = (NCHUNK - NBUF - 1) // NBUF

    def body(i, carry):
        j0 = NBUF + i * NBUF
        for u in range(NBUF):
            j = j0 + u
            b = u
            bg = (b + 1) % NBUF
            wait_idx(bg)
            issue_gather(bg)
            wait_data(b)
            compute(b)
            issue_scatter(b)
            wait_scatter((b + NBUF - 1) % NBUF)
            pl.when(j <= NCHUNK - NBUF)(
                lambda jn=j + 3, bn=(b + 3) % NBUF: issue_loads(jn, bn))
        return carry

    lax.fori_loop(0, NBODY, body, 0)

    for j in range(NBUF + NBODY * NBUF, NCHUNK):
        b = j % NBUF
        if j < NCHUNK - 1:
            bg = (b + 1) % NBUF
            wait_idx(bg)
            issue_gather(bg)
        wait_data(b)
        compute(b)
        issue_scatter(b)
        wait_scatter((b + NBUF - 1) % NBUF)
        if j + 3 <= NCHUNK - 1:
            issue_loads(j + 3, (b + 3) % NBUF)
    wait_scatter((NCHUNK - 1) % NBUF)

    plsc.subcore_barrier()

    pltpu.sync_copy(accum.at[pl.ds(r0, ROWS_PER_TILE)],
                    agg_hbm.at[c, pl.ds(r0, ROWS_PER_TILE)])


def _sc_agg(xs, srco, dst, ea, zeros):
    mesh = plsc.VectorSubcoreMesh(core_axis_name="c", subcore_axis_name="s")
    return pl.kernel(
        _agg_body,
        out_type=jax.ShapeDtypeStruct((NC, N_PAD, DH), jnp.float32),
        mesh=mesh,
        scratch_types=[
            pltpu.VMEM_SHARED((N_PAD, DH), jnp.float32),
            [pltpu.VMEM((CHUNK,), jnp.int32) for _ in range(NBUF)],
            [pltpu.VMEM((CHUNK,), jnp.int32) for _ in range(NBUF)],
            [pltpu.VMEM((CHUNK, DH), jnp.float32) for _ in range(NBUF)],
            [pltpu.VMEM((CHUNK, DH), jnp.float32) for _ in range(NBUF)],
            [pltpu.SemaphoreType.DMA for _ in range(NBUF)],
            [pltpu.SemaphoreType.DMA for _ in range(NBUF)],
            [pltpu.SemaphoreType.DMA for _ in range(NBUF)],
        ],
    )(xs, srco, dst, ea, zeros)


ROW_BLK = 2528


def _mlp_body(eps_ref, x_ref, a0_ref, a1_ref, w1_ref, b1_ref, w2_ref, b2_ref,
              out_ref):
    h = x_ref[...] * (1.0 + eps_ref[0]) + jnp.concatenate(
        [a0_ref[0], a1_ref[0]], axis=1)
    h1 = lax.dot_general(h.astype(jnp.bfloat16),
                         w1_ref[...].astype(jnp.bfloat16),
                         (((1,), (1,)), ((), ())),
                         preferred_element_type=jnp.float32)
    h1 = jnp.maximum(h1 + b1_ref[...], 0.0)
    out_ref[...] = lax.dot_general(h1.astype(jnp.bfloat16),
                                   w2_ref[...].astype(jnp.bfloat16),
                                   (((1,), (1,)), ((), ())),
                                   preferred_element_type=jnp.float32) + b2_ref[...]


def _mlp(x, agg, W1, b1, W2, b2, eps):
    grid = (N_PAD // ROW_BLK,)
    return pl.pallas_call(
        _mlp_body,
        grid=grid,
        in_specs=[
            pl.BlockSpec(memory_space=pltpu.SMEM),
            pl.BlockSpec((ROW_BLK, D), lambda i: (i, 0)),
            pl.BlockSpec((1, ROW_BLK, DH), lambda i: (0, i, 0)),
            pl.BlockSpec((1, ROW_BLK, DH), lambda i: (1, i, 0)),
            pl.BlockSpec((D, D), lambda i: (0, 0)),
            pl.BlockSpec((1, D), lambda i: (0, 0)),
            pl.BlockSpec((D, D), lambda i: (0, 0)),
            pl.BlockSpec((1, D), lambda i: (0, 0)),
        ],
        out_specs=pl.BlockSpec((ROW_BLK, D), lambda i: (i, 0)),
        out_shape=jax.ShapeDtypeStruct((N_NODES, D), jnp.float32),
    )(eps, x, agg, agg, W1, b1, W2, b2)


def kernel(x, edge_index, edge_attr, W1, b1, W2, b2, eps):
    src = edge_index[0].astype(jnp.int32)
    dst = edge_index[1].astype(jnp.int32)
    xs = jnp.concatenate([x[:, :DH], x[:, DH:]], axis=0)
    srco = jnp.concatenate([src, src + N_NODES])
    zeros = jnp.zeros((ROWS_PER_TILE, DH), jnp.float32)
    agg = _sc_agg(xs, srco, dst, edge_attr, zeros)
    eps_s = jnp.reshape(eps, (1,)).astype(jnp.float32)
    return _mlp(x, agg, W1, jnp.reshape(b1, (1, D)), W2,
                jnp.reshape(b2, (1, D)), eps_s)
